# baseline (speedup 1.0000x reference)
import jax
import jax.numpy as jnp
from jax import lax
from jax.experimental import pallas as pl
from jax.experimental.pallas import tpu as pltpu
from jax.experimental.pallas import tpu_sc as plsc

BATCH = 16384
DIM = 32
NUM_WORKERS = 32
B_PER_W = BATCH // NUM_WORKERS
BLOCKS = B_PER_W // 16


def _body(sid_hbm, pid_hbm, s_table, p_table, out_hbm,
          sid_v, pid_v, srows, prows, out_v, sem_s, sem_p):
    num_cores = 2
    wid = lax.axis_index("s") * num_cores + lax.axis_index("c")
    base = wid * B_PER_W

    pltpu.sync_copy(sid_hbm.at[pl.ds(base, B_PER_W)], sid_v)
    pltpu.sync_copy(pid_hbm.at[pl.ds(base, B_PER_W)], pid_v)

    cp_s = pltpu.async_copy(s_table.at[sid_v], srows, sem_s)
    cp_p = pltpu.async_copy(p_table.at[pid_v], prows, sem_p)
    cp_s.wait()
    cp_p.wait()

    lanes = lax.iota(jnp.int32, 16)

    def blk_body(blk, _):
        row_idx = blk * 16 + lanes
        acc = jnp.zeros((16,), jnp.float32)
        for d in range(DIM):
            col = jnp.full((16,), d, jnp.int32)
            sv = plsc.load_gather(srows, [row_idx, col])
            pv = plsc.load_gather(prows, [row_idx, col])
            acc = acc + sv * pv
        out_v[pl.ds(blk * 16, 16)] = acc
        return ()

    lax.fori_loop(0, BLOCKS, blk_body, ())

    pltpu.sync_copy(out_v, out_hbm.at[pl.ds(base, B_PER_W)])


@jax.jit
def kernel(sid, pid, scientist_table, paper_table):
    mesh = plsc.VectorSubcoreMesh(core_axis_name="c", subcore_axis_name="s")
    run = pl.kernel(
        _body,
        out_type=jax.ShapeDtypeStruct((BATCH,), jnp.float32),
        mesh=mesh,
        scratch_types=[
            pltpu.VMEM((B_PER_W,), jnp.int32),
            pltpu.VMEM((B_PER_W,), jnp.int32),
            pltpu.VMEM((B_PER_W, DIM), jnp.float32),
            pltpu.VMEM((B_PER_W, DIM), jnp.float32),
            pltpu.VMEM((B_PER_W,), jnp.float32),
            pltpu.SemaphoreType.DMA,
            pltpu.SemaphoreType.DMA,
        ],
        compiler_params=pltpu.CompilerParams(
            needs_layout_passes=False, use_tc_tiling_on_sc=False),
    )
    return run(sid.astype(jnp.int32), pid.astype(jnp.int32),
               scientist_table, paper_table)
